# speedup vs baseline: 1.0228x; 1.0228x over previous
import jax
import jax.numpy as jnp
from jax.experimental import pallas as pl
from jax.experimental.pallas import tpu as pltpu

_VMEM_LIMIT = 48 * 1024 * 1024


def _norm_kernel(x_ref, scale_ref, bias_ref, o_ref):
    o_ref[...] = x_ref[...] * scale_ref[...] + bias_ref[...]


def kernel(img, mean, std):
    b, T, c, h, w = img.shape
    R = b * T * c
    L = h * w
    x = img.reshape(R, L)

    scale_c = (1.0 / (255.0 * std)).astype(jnp.float32)
    bias_c = (-mean / std).astype(jnp.float32)
    scale = jnp.broadcast_to(scale_c[None, :], (b * T, c)).reshape(R, 1)
    bias = jnp.broadcast_to(bias_c[None, :], (b * T, c)).reshape(R, 1)

    row_tile = 8
    while R % row_tile != 0:
        row_tile -= 1

    out = pl.pallas_call(
        _norm_kernel,
        out_shape=jax.ShapeDtypeStruct((R, L), jnp.float32),
        grid=(R // row_tile,),
        in_specs=[
            pl.BlockSpec((row_tile, L), lambda i: (i, 0)),
            pl.BlockSpec((row_tile, 1), lambda i: (i, 0)),
            pl.BlockSpec((row_tile, 1), lambda i: (i, 0)),
        ],
        out_specs=pl.BlockSpec((row_tile, L), lambda i: (i, 0)),
        compiler_params=pltpu.CompilerParams(
            dimension_semantics=("parallel",), vmem_limit_bytes=_VMEM_LIMIT),
    )(x, scale, bias)
    return out.reshape(b, T, c, h, w)


# speedup vs baseline: 4.0280x; 3.9383x over previous
import jax
import jax.numpy as jnp
from jax.experimental import pallas as pl
from jax.experimental.pallas import tpu as pltpu

_VMEM_LIMIT = 48 * 1024 * 1024


def _norm_kernel(x_ref, scale_ref, bias_ref, o_ref):
    o_ref[...] = x_ref[...] * scale_ref[...] + bias_ref[...]


def kernel(img, mean, std):
    b, T, c, h, w = img.shape
    R = b * T * c
    x = img.reshape(R, h, w)

    scale_c = (1.0 / (255.0 * std)).astype(jnp.float32)
    bias_c = (-mean / std).astype(jnp.float32)
    scale = jnp.broadcast_to(scale_c[None, :], (b * T, c)).reshape(R, 1, 1)
    bias = jnp.broadcast_to(bias_c[None, :], (b * T, c)).reshape(R, 1, 1)

    row_tile = 8
    while R % row_tile != 0:
        row_tile -= 1

    out = pl.pallas_call(
        _norm_kernel,
        out_shape=jax.ShapeDtypeStruct((R, h, w), jnp.float32),
        grid=(R // row_tile,),
        in_specs=[
            pl.BlockSpec((row_tile, h, w), lambda i: (i, 0, 0)),
            pl.BlockSpec((row_tile, 1, 1), lambda i: (i, 0, 0)),
            pl.BlockSpec((row_tile, 1, 1), lambda i: (i, 0, 0)),
        ],
        out_specs=pl.BlockSpec((row_tile, h, w), lambda i: (i, 0, 0)),
        compiler_params=pltpu.CompilerParams(
            dimension_semantics=("parallel",), vmem_limit_bytes=_VMEM_LIMIT),
    )(x, scale, bias)
    return out.reshape(b, T, c, h, w)


# speedup vs baseline: 4.3813x; 1.0877x over previous
import jax
import jax.numpy as jnp
from jax.experimental import pallas as pl
from jax.experimental.pallas import tpu as pltpu

_VMEM_LIMIT = 48 * 1024 * 1024


def _norm_kernel(x_ref, scale_ref, bias_ref, o_ref):
    o_ref[...] = x_ref[...] * scale_ref[...] + bias_ref[...]


def kernel(img, mean, std):
    b, T, c, h, w = img.shape
    R = b * T * c
    x = img.reshape(R, h, w)

    scale_c = (1.0 / (255.0 * std)).astype(jnp.float32)
    bias_c = (-mean / std).astype(jnp.float32)
    scale = jnp.broadcast_to(scale_c[None, :], (b * T, c)).reshape(R, 1, 1)
    bias = jnp.broadcast_to(bias_c[None, :], (b * T, c)).reshape(R, 1, 1)

    row_tile = 16
    while R % row_tile != 0:
        row_tile -= 1

    out = pl.pallas_call(
        _norm_kernel,
        out_shape=jax.ShapeDtypeStruct((R, h, w), jnp.float32),
        grid=(R // row_tile,),
        in_specs=[
            pl.BlockSpec((row_tile, h, w), lambda i: (i, 0, 0)),
            pl.BlockSpec((row_tile, 1, 1), lambda i: (i, 0, 0)),
            pl.BlockSpec((row_tile, 1, 1), lambda i: (i, 0, 0)),
        ],
        out_specs=pl.BlockSpec((row_tile, h, w), lambda i: (i, 0, 0)),
        compiler_params=pltpu.CompilerParams(
            dimension_semantics=("parallel",), vmem_limit_bytes=_VMEM_LIMIT),
    )(x, scale, bias)
    return out.reshape(b, T, c, h, w)


# speedup vs baseline: 4.5492x; 1.0383x over previous
import jax
import jax.numpy as jnp
from jax.experimental import pallas as pl
from jax.experimental.pallas import tpu as pltpu

_VMEM_LIMIT = 48 * 1024 * 1024


def _norm_kernel(x_ref, scale_ref, bias_ref, o_ref):
    o_ref[...] = x_ref[...] * scale_ref[...] + bias_ref[...]


def kernel(img, mean, std):
    b, T, c, h, w = img.shape
    R = b * T * c
    x = img.reshape(R, h, w)

    scale_c = (1.0 / (255.0 * std)).astype(jnp.float32)
    bias_c = (-mean / std).astype(jnp.float32)
    scale = jnp.broadcast_to(scale_c[None, :], (b * T, c)).reshape(R, 1, 1)
    bias = jnp.broadcast_to(bias_c[None, :], (b * T, c)).reshape(R, 1, 1)

    row_tile = 24
    while R % row_tile != 0:
        row_tile -= 1

    out = pl.pallas_call(
        _norm_kernel,
        out_shape=jax.ShapeDtypeStruct((R, h, w), jnp.float32),
        grid=(R // row_tile,),
        in_specs=[
            pl.BlockSpec((row_tile, h, w), lambda i: (i, 0, 0)),
            pl.BlockSpec((row_tile, 1, 1), lambda i: (i, 0, 0)),
            pl.BlockSpec((row_tile, 1, 1), lambda i: (i, 0, 0)),
        ],
        out_specs=pl.BlockSpec((row_tile, h, w), lambda i: (i, 0, 0)),
        compiler_params=pltpu.CompilerParams(
            dimension_semantics=("parallel",), vmem_limit_bytes=_VMEM_LIMIT),
    )(x, scale, bias)
    return out.reshape(b, T, c, h, w)
